# Initial kernel scaffold; baseline (speedup 1.0000x reference)
import jax
import jax.numpy as jnp
from jax.experimental import pallas as pl


def kernel(indices, weight):
    raise NotImplementedError("write your pallas kernel here")


# speedup vs baseline: 2.4891x; 2.4891x over previous
import functools

import jax
import jax.numpy as jnp
from jax import lax
from jax.experimental import pallas as pl
from jax.experimental.pallas import tpu as pltpu
from jax.experimental.pallas import tpu_sc as plsc

NUM_EMB = 1000000
DIM = 16
BATCH = 16384
HIST = 200
B_TOTAL = BATCH * HIST

_info = plsc.get_sparse_core_info()
NC, NS = _info.num_cores, _info.num_subcores
NW = NC * NS
B_PER_W = B_TOTAL // NW
CHUNK = 2048
G = B_PER_W // CHUNK

_mesh = plsc.VectorSubcoreMesh(core_axis_name="c", subcore_axis_name="s")


@functools.partial(
    pl.kernel,
    mesh=_mesh,
    out_type=jax.ShapeDtypeStruct((B_TOTAL, DIM), jnp.float32),
    scratch_types=[
        pltpu.VMEM((CHUNK,), jnp.int32),
        pltpu.VMEM((CHUNK, DIM), jnp.float32),
        pltpu.SemaphoreType.DMA,
    ],
    compiler_params=pltpu.CompilerParams(use_tc_tiling_on_sc=False),
)
def _gather_kernel(table_hbm, idx_hbm, out_hbm, idx_v, rows_v, sem):
    wid = lax.axis_index("s") * NC + lax.axis_index("c")
    base = wid * B_PER_W

    def chunk_body(g, carry):
        off = base + g * CHUNK
        pltpu.sync_copy(idx_hbm.at[pl.ds(off, CHUNK)], idx_v)
        pltpu.async_copy(table_hbm.at[idx_v], rows_v, sem).wait()
        pltpu.sync_copy(rows_v, out_hbm.at[pl.ds(off, CHUNK)])
        return carry

    lax.fori_loop(0, G, chunk_body, 0)


def kernel(indices, weight):
    idx_flat = indices.reshape(B_TOTAL).astype(jnp.int32)
    out = _gather_kernel(weight, idx_flat)
    return out.reshape(BATCH, HIST, DIM)


# speedup vs baseline: 5.0874x; 2.0438x over previous
import functools

import jax
import jax.numpy as jnp
from jax import lax
from jax.experimental import pallas as pl
from jax.experimental.pallas import tpu as pltpu
from jax.experimental.pallas import tpu_sc as plsc

NUM_EMB = 1000000
DIM = 16
BATCH = 16384
HIST = 200

_info = plsc.get_sparse_core_info()
NC, NS = _info.num_cores, _info.num_subcores
NW = NC * NS

TT = HIST // 8
BT = BATCH // 128
UNITS = TT * BT
U_PER_W = UNITS // NW

_mesh = plsc.VectorSubcoreMesh(core_axis_name="c", subcore_axis_name="s")


@functools.partial(
    pl.kernel,
    mesh=_mesh,
    out_type=jax.ShapeDtypeStruct((HIST, 2, BT, 8, 128), jnp.float32),
    scratch_types=[
        pltpu.VMEM((1024,), jnp.int32),
        pltpu.VMEM((1024,), jnp.int32),
        pltpu.VMEM((1024, DIM), jnp.float32),
        pltpu.VMEM((1024, DIM), jnp.float32),
        pltpu.VMEM((8, 2, 8, 128), jnp.float32),
        pltpu.VMEM((8, 2, 8, 128), jnp.float32),
        pltpu.SemaphoreType.DMA,
        pltpu.SemaphoreType.DMA,
        pltpu.SemaphoreType.DMA,
        pltpu.SemaphoreType.DMA,
        pltpu.SemaphoreType.DMA,
        pltpu.SemaphoreType.DMA,
    ],
    compiler_params=pltpu.CompilerParams(
        use_tc_tiling_on_sc=False, needs_layout_passes=False
    ),
)
def _gather_kernel(
    w_hbm,
    idxp_hbm,
    outp_hbm,
    idx_v0,
    idx_v1,
    rows_v0,
    rows_v1,
    ob0,
    ob1,
    isem0,
    isem1,
    gsem0,
    gsem1,
    osem0,
    osem1,
):
    idx_v = (idx_v0, idx_v1)
    rows_v = (rows_v0, rows_v1)
    ob = (ob0, ob1)
    isem = (isem0, isem1)
    gsem = (gsem0, gsem1)
    osem = (osem0, osem1)

    wid = lax.axis_index("s") * NC + lax.axis_index("c")
    u0 = wid * U_PER_W
    lane_iota = lax.iota(jnp.int32, 16)
    dcol = [jnp.full((16,), d, jnp.int32) for d in range(DIM)]

    def start_idx(u, b):
        tt = u // BT
        bt = u % BT
        pltpu.async_copy(idxp_hbm.at[tt, bt], idx_v[b], isem[b])

    def wait_idx(b):
        pltpu.make_async_copy(idxp_hbm.at[0, 0], idx_v[b], isem[b]).wait()

    def start_gather(b):
        pltpu.async_copy(w_hbm.at[idx_v[b]], rows_v[b], gsem[b])

    def wait_gather(b):
        pltpu.make_async_copy(
            w_hbm.at[idx_v[b]], rows_v[b], gsem[b]
        ).wait()

    def start_out(u, b):
        tt = u // BT
        bt = u % BT
        pltpu.async_copy(ob[b], outp_hbm.at[pl.ds(tt * 8, 8), :, bt], osem[b])

    def wait_out(b):
        pltpu.make_async_copy(
            ob[b], outp_hbm.at[pl.ds(0, 8), :, 0], osem[b]
        ).wait()

    def transpose(b):
        rows = rows_v[b]
        obuf = ob[b]

        def tbody(g, carry):
            ts = g // 8
            r0 = g % 8
            iv_row = lane_iota + g * 16
            for d in range(DIM):
                v = plsc.load_gather(rows, [iv_row, dcol[d]])
                obuf[ts, d // 8, d % 8, pl.ds(r0 * 16, 16)] = v
            return carry

        lax.fori_loop(0, 64, tbody, 0)

    start_idx(u0, 0)
    start_idx(u0 + 1, 1)
    wait_idx(0)
    start_gather(0)

    def outer(o, carry):
        for b in (0, 1):
            k = o * 2 + b
            nb = 1 - b
            wait_gather(b)

            @pl.when(k + 1 < U_PER_W)
            def _():
                wait_idx(nb)
                start_gather(nb)

            @pl.when(k + 2 < U_PER_W)
            def _():
                start_idx(u0 + k + 2, b)

            @pl.when(k >= 2)
            def _():
                wait_out(b)

            transpose(b)
            start_out(u0 + k, b)
        return carry

    lax.fori_loop(0, U_PER_W // 2, outer, 0)
    wait_out(0)
    wait_out(1)


def kernel(indices, weight):
    idx_phys = (
        indices.reshape(BT, 128, TT, 8)
        .transpose(2, 0, 3, 1)
        .reshape(TT, BT, 1024)
    )
    out_phys = _gather_kernel(weight, idx_phys)
    return out_phys.transpose(2, 4, 0, 1, 3).reshape(BATCH, HIST, DIM)


# speedup vs baseline: 8.5068x; 1.6721x over previous
import functools

import jax
import jax.numpy as jnp
from jax import lax
from jax.experimental import pallas as pl
from jax.experimental.pallas import tpu as pltpu
from jax.experimental.pallas import tpu_sc as plsc

NUM_EMB = 1000000
DIM = 16
BATCH = 16384
HIST = 200

_info = plsc.get_sparse_core_info()
NC, NS = _info.num_cores, _info.num_subcores
NW = NC * NS

TT = HIST // 8
BT = BATCH // 128
UNITS = TT * BT
U_PER_W = UNITS // NW

_mesh = plsc.VectorSubcoreMesh(core_axis_name="c", subcore_axis_name="s")


@functools.partial(
    pl.kernel,
    mesh=_mesh,
    out_type=jax.ShapeDtypeStruct((HIST, 2, BT, 8, 128), jnp.float32),
    scratch_types=[
        pltpu.VMEM((1024,), jnp.int32),
        pltpu.VMEM((1024,), jnp.int32),
        pltpu.VMEM((1024, DIM), jnp.float32),
        pltpu.VMEM((1024, DIM), jnp.float32),
        pltpu.VMEM((8, 2, 8, 128), jnp.float32),
        pltpu.VMEM((8, 2, 8, 128), jnp.float32),
        pltpu.SemaphoreType.DMA,
        pltpu.SemaphoreType.DMA,
        pltpu.SemaphoreType.DMA,
        pltpu.SemaphoreType.DMA,
        pltpu.SemaphoreType.DMA,
        pltpu.SemaphoreType.DMA,
    ],
    compiler_params=pltpu.CompilerParams(
        use_tc_tiling_on_sc=False, needs_layout_passes=False
    ),
)
def _gather_kernel(
    w_hbm,
    idxp_hbm,
    outp_hbm,
    idx_v0,
    idx_v1,
    rows_v0,
    rows_v1,
    ob0,
    ob1,
    isem0,
    isem1,
    gsem0,
    gsem1,
    osem0,
    osem1,
):
    idx_v = (idx_v0, idx_v1)
    rows_v = (rows_v0, rows_v1)
    ob = (ob0, ob1)
    isem = (isem0, isem1)
    gsem = (gsem0, gsem1)
    osem = (osem0, osem1)

    wid = lax.axis_index("s") * NC + lax.axis_index("c")
    u0 = wid * U_PER_W
    lane_iota = lax.iota(jnp.int32, 16)
    dcol = [jnp.full((16,), d, jnp.int32) for d in range(DIM)]

    def start_idx(u, b):
        tt = u // BT
        bt = u % BT
        pltpu.async_copy(idxp_hbm.at[tt, bt], idx_v[b], isem[b])

    def wait_idx(b):
        pltpu.make_async_copy(idxp_hbm.at[0, 0], idx_v[b], isem[b]).wait()

    def start_gather(b):
        pltpu.async_copy(w_hbm.at[idx_v[b]], rows_v[b], gsem[b])

    def wait_gather(b):
        pltpu.make_async_copy(
            w_hbm.at[idx_v[b]], rows_v[b], gsem[b]
        ).wait()

    def start_out(u, b):
        tt = u // BT
        bt = u % BT
        pltpu.async_copy(ob[b], outp_hbm.at[pl.ds(tt * 8, 8), :, bt], osem[b])

    def wait_out(b):
        pltpu.make_async_copy(
            ob[b], outp_hbm.at[pl.ds(0, 8), :, 0], osem[b]
        ).wait()

    def transpose(b):
        rows = rows_v[b]
        obuf = ob[b]

        @plsc.parallel_loop(0, 64, 1, unroll=2)
        def tbody(g):
            ts = g // 8
            r0 = g % 8
            iv_row = lane_iota + g * 16
            for d in range(DIM):
                v = plsc.load_gather(rows, [iv_row, dcol[d]])
                obuf[ts, d // 8, d % 8, pl.ds(r0 * 16, 16)] = v

    start_idx(u0, 0)
    start_idx(u0 + 1, 1)
    wait_idx(0)
    start_gather(0)

    def outer(o, carry):
        for b in (0, 1):
            k = o * 2 + b
            nb = 1 - b
            wait_gather(b)

            @pl.when(k + 1 < U_PER_W)
            def _():
                wait_idx(nb)
                start_gather(nb)

            @pl.when(k + 2 < U_PER_W)
            def _():
                start_idx(u0 + k + 2, b)

            @pl.when(k >= 2)
            def _():
                wait_out(b)

            transpose(b)
            start_out(u0 + k, b)
        return carry

    lax.fori_loop(0, U_PER_W // 2, outer, 0)
    wait_out(0)
    wait_out(1)


def kernel(indices, weight):
    idx_phys = (
        indices.reshape(BT, 128, TT, 8)
        .transpose(2, 0, 3, 1)
        .reshape(TT, BT, 1024)
    )
    out_phys = _gather_kernel(weight, idx_phys)
    return out_phys.transpose(2, 4, 0, 1, 3).reshape(BATCH, HIST, DIM)


# speedup vs baseline: 11.5193x; 1.3541x over previous
import functools

import jax
import jax.numpy as jnp
from jax import lax
from jax.experimental import pallas as pl
from jax.experimental.pallas import tpu as pltpu
from jax.experimental.pallas import tpu_sc as plsc

NUM_EMB = 1000000
DIM = 16
BATCH = 16384
HIST = 200

_info = plsc.get_sparse_core_info()
NC, NS = _info.num_cores, _info.num_subcores
NW = NC * NS

TT = HIST // 8
BT = BATCH // 128
UNITS = TT * BT
U_PER_W = UNITS // NW

VPAD = 64
VT = (NUM_EMB + VPAD) // 128
A_PER_W = -(-VT // NW)

_mesh = plsc.VectorSubcoreMesh(core_axis_name="c", subcore_axis_name="s")


@functools.partial(
    pl.kernel,
    mesh=_mesh,
    out_type=jax.ShapeDtypeStruct((VT * 128, DIM), jnp.float32),
    scratch_types=[
        pltpu.VMEM((DIM, 128), jnp.float32),
        pltpu.VMEM((DIM, 128), jnp.float32),
        pltpu.VMEM((128, DIM), jnp.float32),
        pltpu.VMEM((128, DIM), jnp.float32),
        pltpu.SemaphoreType.DMA,
        pltpu.SemaphoreType.DMA,
        pltpu.SemaphoreType.DMA,
        pltpu.SemaphoreType.DMA,
    ],
    compiler_params=pltpu.CompilerParams(
        use_tc_tiling_on_sc=False, needs_layout_passes=False
    ),
)
def _wt_kernel(wphys_hbm, wlin_hbm, tb0, tb1, lb0, lb1, ts0, ts1, os0, os1):
    tb = (tb0, tb1)
    lb = (lb0, lb1)
    tsem = (ts0, ts1)
    osem = (os0, os1)

    wid = lax.axis_index("s") * NC + lax.axis_index("c")
    base = wid * A_PER_W
    nmy = jnp.clip(VT - base, 0, A_PER_W)
    lane_iota = lax.iota(jnp.int32, 16)
    zero16 = jnp.zeros((16,), jnp.int32)

    def start_in(k, b):
        pltpu.async_copy(
            wphys_hbm.at[0, base + k], tb[b].at[pl.ds(0, 8), :], tsem[b]
        )
        pltpu.async_copy(
            wphys_hbm.at[1, base + k], tb[b].at[pl.ds(8, 8), :], tsem[b]
        )

    def wait_in(b):
        pltpu.make_async_copy(
            wphys_hbm.at[0, 0], tb[b].at[pl.ds(0, 8), :], tsem[b]
        ).wait()
        pltpu.make_async_copy(
            wphys_hbm.at[1, 0], tb[b].at[pl.ds(8, 8), :], tsem[b]
        ).wait()

    def start_out(k, b):
        pltpu.async_copy(
            lb[b], wlin_hbm.at[pl.ds((base + k) * 128, 128)], osem[b]
        )

    def wait_out(b):
        pltpu.make_async_copy(
            lb[b], wlin_hbm.at[pl.ds(0, 128)], osem[b]
        ).wait()

    def transpose(b):
        tbuf = tb[b]
        lbuf = lb[b]

        @plsc.parallel_loop(0, 128, 1, unroll=4)
        def tbody(r):
            v = plsc.load_gather(tbuf, [lane_iota, zero16 + r])
            lbuf[r] = v

    @pl.when(nmy > 0)
    def _():
        start_in(0, 0)

    @pl.when(nmy > 1)
    def _():
        start_in(1, 1)

    def outer(o, carry):
        for b in (0, 1):
            k = o * 2 + b

            @pl.when(k < nmy)
            def _():
                wait_in(b)

                @pl.when(k >= 2)
                def _():
                    wait_out(b)

                transpose(b)

                @pl.when(k + 2 < nmy)
                def _():
                    start_in(k + 2, b)

                start_out(k, b)
        return carry

    lax.fori_loop(0, (A_PER_W + 1) // 2, outer, 0)

    @pl.when(nmy > 0)
    def _():
        wait_out(0)

    @pl.when(nmy > 1)
    def _():
        wait_out(1)


@functools.partial(
    pl.kernel,
    mesh=_mesh,
    out_type=jax.ShapeDtypeStruct((HIST, 2, BT, 8, 128), jnp.float32),
    scratch_types=[
        pltpu.VMEM((1024,), jnp.int32),
        pltpu.VMEM((1024,), jnp.int32),
        pltpu.VMEM((1024, DIM), jnp.float32),
        pltpu.VMEM((1024, DIM), jnp.float32),
        pltpu.VMEM((8, 2, 8, 128), jnp.float32),
        pltpu.VMEM((8, 2, 8, 128), jnp.float32),
        pltpu.SemaphoreType.DMA,
        pltpu.SemaphoreType.DMA,
        pltpu.SemaphoreType.DMA,
        pltpu.SemaphoreType.DMA,
        pltpu.SemaphoreType.DMA,
        pltpu.SemaphoreType.DMA,
    ],
    compiler_params=pltpu.CompilerParams(
        use_tc_tiling_on_sc=False, needs_layout_passes=False
    ),
)
def _gather_kernel(
    w_hbm,
    idxp_hbm,
    outp_hbm,
    idx_v0,
    idx_v1,
    rows_v0,
    rows_v1,
    ob0,
    ob1,
    isem0,
    isem1,
    gsem0,
    gsem1,
    osem0,
    osem1,
):
    idx_v = (idx_v0, idx_v1)
    rows_v = (rows_v0, rows_v1)
    ob = (ob0, ob1)
    isem = (isem0, isem1)
    gsem = (gsem0, gsem1)
    osem = (osem0, osem1)

    wid = lax.axis_index("s") * NC + lax.axis_index("c")
    u0 = wid * U_PER_W
    lane_iota = lax.iota(jnp.int32, 16)
    dcol = [jnp.full((16,), d, jnp.int32) for d in range(DIM)]

    def start_idx(u, b):
        tt = u // BT
        bt = u % BT
        pltpu.async_copy(idxp_hbm.at[tt, bt], idx_v[b], isem[b])

    def wait_idx(b):
        pltpu.make_async_copy(idxp_hbm.at[0, 0], idx_v[b], isem[b]).wait()

    def start_gather(b):
        pltpu.async_copy(w_hbm.at[idx_v[b]], rows_v[b], gsem[b])

    def wait_gather(b):
        pltpu.make_async_copy(
            w_hbm.at[idx_v[b]], rows_v[b], gsem[b]
        ).wait()

    def start_out(u, b):
        tt = u // BT
        bt = u % BT
        pltpu.async_copy(ob[b], outp_hbm.at[pl.ds(tt * 8, 8), :, bt], osem[b])

    def wait_out(b):
        pltpu.make_async_copy(
            ob[b], outp_hbm.at[pl.ds(0, 8), :, 0], osem[b]
        ).wait()

    def transpose(b):
        rows = rows_v[b]
        obuf = ob[b]

        @plsc.parallel_loop(0, 64, 1, unroll=2)
        def tbody(g):
            ts = g // 8
            r0 = g % 8
            iv_row = lane_iota + g * 16
            for d in range(DIM):
                v = plsc.load_gather(rows, [iv_row, dcol[d]])
                obuf[ts, d // 8, d % 8, pl.ds(r0 * 16, 16)] = v

    start_idx(u0, 0)
    start_idx(u0 + 1, 1)
    wait_idx(0)
    start_gather(0)

    def outer(o, carry):
        for b in (0, 1):
            k = o * 2 + b
            nb = 1 - b
            wait_gather(b)

            @pl.when(k + 1 < U_PER_W)
            def _():
                wait_idx(nb)
                start_gather(nb)

            @pl.when(k + 2 < U_PER_W)
            def _():
                start_idx(u0 + k + 2, b)

            @pl.when(k >= 2)
            def _():
                wait_out(b)

            transpose(b)
            start_out(u0 + k, b)
        return carry

    lax.fori_loop(0, U_PER_W // 2, outer, 0)
    wait_out(0)
    wait_out(1)


def kernel(indices, weight):
    wpad = jnp.concatenate(
        [weight, jnp.zeros((VPAD, DIM), jnp.float32)], axis=0
    )
    wphys = wpad.reshape(VT, 128, 2, 8).transpose(2, 0, 3, 1)
    w_lin = _wt_kernel(wphys)
    idx_phys = (
        indices.reshape(BT, 128, TT, 8)
        .transpose(2, 0, 3, 1)
        .reshape(TT, BT, 1024)
    )
    out_phys = _gather_kernel(w_lin, idx_phys)
    return out_phys.transpose(2, 4, 0, 1, 3).reshape(BATCH, HIST, DIM)


# speedup vs baseline: 11.5337x; 1.0013x over previous
import functools

import jax
import jax.numpy as jnp
from jax import lax
from jax.experimental import pallas as pl
from jax.experimental.pallas import tpu as pltpu
from jax.experimental.pallas import tpu_sc as plsc

NUM_EMB = 1000000
DIM = 16
BATCH = 16384
HIST = 200

_info = plsc.get_sparse_core_info()
NC, NS = _info.num_cores, _info.num_subcores
NW = NC * NS

TT = HIST // 8
BT = BATCH // 128
UNITS = TT * BT
U_PER_W = UNITS // NW

VT = 7816
VPAD = VT * 128 - NUM_EMB
VT4 = VT // 4
A_PER_W = -(-VT4 // NW)

_mesh = plsc.VectorSubcoreMesh(core_axis_name="c", subcore_axis_name="s")


@functools.partial(
    pl.kernel,
    mesh=_mesh,
    out_type=jax.ShapeDtypeStruct((VT * 128, DIM), jnp.float32),
    scratch_types=[
        pltpu.VMEM((4, DIM, 128), jnp.float32),
        pltpu.VMEM((4, DIM, 128), jnp.float32),
        pltpu.VMEM((512, DIM), jnp.float32),
        pltpu.VMEM((512, DIM), jnp.float32),
        pltpu.SemaphoreType.DMA,
        pltpu.SemaphoreType.DMA,
        pltpu.SemaphoreType.DMA,
        pltpu.SemaphoreType.DMA,
    ],
    compiler_params=pltpu.CompilerParams(
        use_tc_tiling_on_sc=False, needs_layout_passes=False
    ),
)
def _wt_kernel(wphys_hbm, wlin_hbm, tb0, tb1, lb0, lb1, ts0, ts1, os0, os1):
    tb = (tb0, tb1)
    lb = (lb0, lb1)
    tsem = (ts0, ts1)
    osem = (os0, os1)

    wid = lax.axis_index("s") * NC + lax.axis_index("c")
    base = wid * A_PER_W
    nmy = jnp.clip(VT4 - base, 0, A_PER_W)
    lane_iota = lax.iota(jnp.int32, 16)
    zero16 = jnp.zeros((16,), jnp.int32)

    def start_in(k, b):
        vt0 = (base + k) * 4
        pltpu.async_copy(
            wphys_hbm.at[0, pl.ds(vt0, 4)],
            tb[b].at[:, pl.ds(0, 8), :],
            tsem[b],
        )
        pltpu.async_copy(
            wphys_hbm.at[1, pl.ds(vt0, 4)],
            tb[b].at[:, pl.ds(8, 8), :],
            tsem[b],
        )

    def wait_in(b):
        pltpu.make_async_copy(
            wphys_hbm.at[0, pl.ds(0, 4)], tb[b].at[:, pl.ds(0, 8), :], tsem[b]
        ).wait()
        pltpu.make_async_copy(
            wphys_hbm.at[1, pl.ds(0, 4)], tb[b].at[:, pl.ds(8, 8), :], tsem[b]
        ).wait()

    def start_out(k, b):
        pltpu.async_copy(
            lb[b], wlin_hbm.at[pl.ds((base + k) * 512, 512)], osem[b]
        )

    def wait_out(b):
        pltpu.make_async_copy(
            lb[b], wlin_hbm.at[pl.ds(0, 512)], osem[b]
        ).wait()

    def transpose(b):
        lbuf = lb[b]
        for vti in range(4):
            tbuf = tb[b].at[vti]

            @plsc.parallel_loop(0, 128, 1, unroll=4)
            def tbody(r):
                v = plsc.load_gather(tbuf, [lane_iota, zero16 + r])
                lbuf[vti * 128 + r] = v

    @pl.when(nmy > 0)
    def _():
        start_in(0, 0)

    @pl.when(nmy > 1)
    def _():
        start_in(1, 1)

    def outer(o, carry):
        for b in (0, 1):
            k = o * 2 + b

            @pl.when(k < nmy)
            def _():
                wait_in(b)

                @pl.when(k >= 2)
                def _():
                    wait_out(b)

                transpose(b)

                @pl.when(k + 2 < nmy)
                def _():
                    start_in(k + 2, b)

                start_out(k, b)
        return carry

    lax.fori_loop(0, (A_PER_W + 1) // 2, outer, 0)

    @pl.when(nmy > 0)
    def _():
        wait_out(0)

    @pl.when(nmy > 1)
    def _():
        wait_out(1)


@functools.partial(
    pl.kernel,
    mesh=_mesh,
    out_type=jax.ShapeDtypeStruct((HIST, 2, BT, 8, 128), jnp.float32),
    scratch_types=[
        pltpu.VMEM((1024,), jnp.int32),
        pltpu.VMEM((1024,), jnp.int32),
        pltpu.VMEM((1024, DIM), jnp.float32),
        pltpu.VMEM((1024, DIM), jnp.float32),
        pltpu.VMEM((8, 2, 8, 128), jnp.float32),
        pltpu.VMEM((8, 2, 8, 128), jnp.float32),
        pltpu.SemaphoreType.DMA,
        pltpu.SemaphoreType.DMA,
        pltpu.SemaphoreType.DMA,
        pltpu.SemaphoreType.DMA,
        pltpu.SemaphoreType.DMA,
        pltpu.SemaphoreType.DMA,
    ],
    compiler_params=pltpu.CompilerParams(
        use_tc_tiling_on_sc=False, needs_layout_passes=False
    ),
)
def _gather_kernel(
    w_hbm,
    idxp_hbm,
    outp_hbm,
    idx_v0,
    idx_v1,
    rows_v0,
    rows_v1,
    ob0,
    ob1,
    isem0,
    isem1,
    gsem0,
    gsem1,
    osem0,
    osem1,
):
    idx_v = (idx_v0, idx_v1)
    rows_v = (rows_v0, rows_v1)
    ob = (ob0, ob1)
    isem = (isem0, isem1)
    gsem = (gsem0, gsem1)
    osem = (osem0, osem1)

    wid = lax.axis_index("s") * NC + lax.axis_index("c")
    u0 = wid * U_PER_W
    lane_iota = lax.iota(jnp.int32, 16)
    dcol = [jnp.full((16,), d, jnp.int32) for d in range(DIM)]

    def start_idx(u, b):
        tt = u // BT
        bt = u % BT
        pltpu.async_copy(idxp_hbm.at[tt, bt], idx_v[b], isem[b])

    def wait_idx(b):
        pltpu.make_async_copy(idxp_hbm.at[0, 0], idx_v[b], isem[b]).wait()

    def start_gather(b):
        pltpu.async_copy(w_hbm.at[idx_v[b]], rows_v[b], gsem[b])

    def wait_gather(b):
        pltpu.make_async_copy(
            w_hbm.at[idx_v[b]], rows_v[b], gsem[b]
        ).wait()

    def start_out(u, b):
        tt = u // BT
        bt = u % BT
        pltpu.async_copy(ob[b], outp_hbm.at[pl.ds(tt * 8, 8), :, bt], osem[b])

    def wait_out(b):
        pltpu.make_async_copy(
            ob[b], outp_hbm.at[pl.ds(0, 8), :, 0], osem[b]
        ).wait()

    def transpose(b):
        rows = rows_v[b]
        obuf = ob[b]

        @plsc.parallel_loop(0, 64, 1, unroll=2)
        def tbody(g):
            ts = g // 8
            r0 = g % 8
            iv_row = lane_iota + g * 16
            for d in range(DIM):
                v = plsc.load_gather(rows, [iv_row, dcol[d]])
                obuf[ts, d // 8, d % 8, pl.ds(r0 * 16, 16)] = v

    start_idx(u0, 0)
    start_idx(u0 + 1, 1)
    wait_idx(0)
    start_gather(0)

    def outer(o, carry):
        for b in (0, 1):
            k = o * 2 + b
            nb = 1 - b
            wait_gather(b)

            @pl.when(k + 1 < U_PER_W)
            def _():
                wait_idx(nb)
                start_gather(nb)

            @pl.when(k + 2 < U_PER_W)
            def _():
                start_idx(u0 + k + 2, b)

            @pl.when(k >= 2)
            def _():
                wait_out(b)

            transpose(b)
            start_out(u0 + k, b)
        return carry

    lax.fori_loop(0, U_PER_W // 2, outer, 0)
    wait_out(0)
    wait_out(1)


def kernel(indices, weight):
    wpad = jnp.concatenate(
        [weight, jnp.zeros((VPAD, DIM), jnp.float32)], axis=0
    )
    wphys = wpad.reshape(VT, 128, 2, 8).transpose(2, 0, 3, 1)
    w_lin = _wt_kernel(wphys)
    idx_phys = (
        indices.reshape(BT, 128, TT, 8)
        .transpose(2, 0, 3, 1)
        .reshape(TT, BT, 1024)
    )
    out_phys = _gather_kernel(w_lin, idx_phys)
    return out_phys.transpose(2, 4, 0, 1, 3).reshape(BATCH, HIST, DIM)


# speedup vs baseline: 15.7165x; 1.3627x over previous
import functools

import jax
import jax.numpy as jnp
from jax import lax
from jax.experimental import pallas as pl
from jax.experimental.pallas import tpu as pltpu
from jax.experimental.pallas import tpu_sc as plsc

NUM_EMB = 1000000
DIM = 16
BATCH = 16384
HIST = 200

_info = plsc.get_sparse_core_info()
NC, NS = _info.num_cores, _info.num_subcores
NW = NC * NS

TT = HIST // 8
BT = BATCH // 128
UNITS = TT * BT
U_PER_W = UNITS // NW

VT = 7816
VPAD = VT * 128 - NUM_EMB
VT4 = VT // 4
A_PER_W = -(-VT4 // NW)

_mesh = plsc.VectorSubcoreMesh(core_axis_name="c", subcore_axis_name="s")


@functools.partial(
    pl.kernel,
    mesh=_mesh,
    out_type=jax.ShapeDtypeStruct((VT * 128, DIM), jnp.float32),
    scratch_types=[
        pltpu.VMEM((4, DIM, 136), jnp.float32),
        pltpu.VMEM((4, DIM, 136), jnp.float32),
        pltpu.VMEM((512, DIM), jnp.float32),
        pltpu.VMEM((512, DIM), jnp.float32),
        pltpu.SemaphoreType.DMA,
        pltpu.SemaphoreType.DMA,
        pltpu.SemaphoreType.DMA,
        pltpu.SemaphoreType.DMA,
    ],
    compiler_params=pltpu.CompilerParams(
        use_tc_tiling_on_sc=False, needs_layout_passes=False
    ),
)
def _wt_kernel(wphys_hbm, wlin_hbm, tb0, tb1, lb0, lb1, ts0, ts1, os0, os1):
    tb = (tb0, tb1)
    lb = (lb0, lb1)
    tsem = (ts0, ts1)
    osem = (os0, os1)

    wid = lax.axis_index("s") * NC + lax.axis_index("c")
    base = wid * A_PER_W
    nmy = jnp.clip(VT4 - base, 0, A_PER_W)
    lane_iota = lax.iota(jnp.int32, 16)
    zero16 = jnp.zeros((16,), jnp.int32)

    def start_in(k, b):
        vt0 = (base + k) * 4
        pltpu.async_copy(
            wphys_hbm.at[0, pl.ds(vt0, 4)],
            tb[b].at[:, pl.ds(0, 8), pl.ds(0, 128)],
            tsem[b],
        )
        pltpu.async_copy(
            wphys_hbm.at[1, pl.ds(vt0, 4)],
            tb[b].at[:, pl.ds(8, 8), pl.ds(0, 128)],
            tsem[b],
        )

    def wait_in(b):
        pltpu.make_async_copy(
            wphys_hbm.at[0, pl.ds(0, 4)],
            tb[b].at[:, pl.ds(0, 8), pl.ds(0, 128)],
            tsem[b],
        ).wait()
        pltpu.make_async_copy(
            wphys_hbm.at[1, pl.ds(0, 4)],
            tb[b].at[:, pl.ds(8, 8), pl.ds(0, 128)],
            tsem[b],
        ).wait()

    def start_out(k, b):
        pltpu.async_copy(
            lb[b], wlin_hbm.at[pl.ds((base + k) * 512, 512)], osem[b]
        )

    def wait_out(b):
        pltpu.make_async_copy(
            lb[b], wlin_hbm.at[pl.ds(0, 512)], osem[b]
        ).wait()

    def transpose(b):
        lbuf = lb[b]
        for vti in range(4):
            tbuf = tb[b].at[vti]

            @plsc.parallel_loop(0, 128, 1, unroll=4)
            def tbody(r):
                v = plsc.load_gather(tbuf, [lane_iota, zero16 + r])
                lbuf[vti * 128 + r] = v

    @pl.when(nmy > 0)
    def _():
        start_in(0, 0)

    @pl.when(nmy > 1)
    def _():
        start_in(1, 1)

    def outer(o, carry):
        for b in (0, 1):
            k = o * 2 + b

            @pl.when(k < nmy)
            def _():
                wait_in(b)

                @pl.when(k >= 2)
                def _():
                    wait_out(b)

                transpose(b)

                @pl.when(k + 2 < nmy)
                def _():
                    start_in(k + 2, b)

                start_out(k, b)
        return carry

    lax.fori_loop(0, (A_PER_W + 1) // 2, outer, 0)

    @pl.when(nmy > 0)
    def _():
        wait_out(0)

    @pl.when(nmy > 1)
    def _():
        wait_out(1)


@functools.partial(
    pl.kernel,
    mesh=_mesh,
    out_type=jax.ShapeDtypeStruct((HIST, 2, BT, 8, 128), jnp.float32),
    scratch_types=[
        pltpu.VMEM((1024,), jnp.int32),
        pltpu.VMEM((1024,), jnp.int32),
        pltpu.VMEM((1024, DIM), jnp.float32),
        pltpu.VMEM((1024, DIM), jnp.float32),
        pltpu.VMEM((8, 2, 8, 128), jnp.float32),
        pltpu.VMEM((8, 2, 8, 128), jnp.float32),
        pltpu.SemaphoreType.DMA,
        pltpu.SemaphoreType.DMA,
        pltpu.SemaphoreType.DMA,
        pltpu.SemaphoreType.DMA,
        pltpu.SemaphoreType.DMA,
        pltpu.SemaphoreType.DMA,
    ],
    compiler_params=pltpu.CompilerParams(
        use_tc_tiling_on_sc=False, needs_layout_passes=False
    ),
)
def _gather_kernel(
    w_hbm,
    idxp_hbm,
    outp_hbm,
    idx_v0,
    idx_v1,
    rows_v0,
    rows_v1,
    ob0,
    ob1,
    isem0,
    isem1,
    gsem0,
    gsem1,
    osem0,
    osem1,
):
    idx_v = (idx_v0, idx_v1)
    rows_v = (rows_v0, rows_v1)
    ob = (ob0, ob1)
    isem = (isem0, isem1)
    gsem = (gsem0, gsem1)
    osem = (osem0, osem1)

    wid = lax.axis_index("s") * NC + lax.axis_index("c")
    u0 = wid * U_PER_W
    lane_iota = lax.iota(jnp.int32, 16)
    dcol = [jnp.full((16,), d, jnp.int32) for d in range(DIM)]

    def start_idx(u, b):
        tt = u // BT
        bt = u % BT
        pltpu.async_copy(idxp_hbm.at[tt, bt], idx_v[b], isem[b])

    def wait_idx(b):
        pltpu.make_async_copy(idxp_hbm.at[0, 0], idx_v[b], isem[b]).wait()

    def start_gather(b):
        pltpu.async_copy(w_hbm.at[idx_v[b]], rows_v[b], gsem[b])

    def wait_gather(b):
        pltpu.make_async_copy(
            w_hbm.at[idx_v[b]], rows_v[b], gsem[b]
        ).wait()

    def start_out(u, b):
        tt = u // BT
        bt = u % BT
        pltpu.async_copy(ob[b], outp_hbm.at[pl.ds(tt * 8, 8), :, bt], osem[b])

    def wait_out(b):
        pltpu.make_async_copy(
            ob[b], outp_hbm.at[pl.ds(0, 8), :, 0], osem[b]
        ).wait()

    def transpose(b):
        rows = rows_v[b]
        obuf = ob[b]

        @plsc.parallel_loop(0, 64, 1, unroll=2)
        def tbody(g):
            ts = g // 8
            r0 = g % 8
            iv_row = lane_iota + g * 16
            for d in range(DIM):
                v = plsc.load_gather(rows, [iv_row, dcol[d]])
                obuf[ts, d // 8, d % 8, pl.ds(r0 * 16, 16)] = v

    start_idx(u0, 0)
    start_idx(u0 + 1, 1)
    wait_idx(0)
    start_gather(0)

    def outer(o, carry):
        for b in (0, 1):
            k = o * 2 + b
            nb = 1 - b
            wait_gather(b)

            @pl.when(k + 1 < U_PER_W)
            def _():
                wait_idx(nb)
                start_gather(nb)

            @pl.when(k + 2 < U_PER_W)
            def _():
                start_idx(u0 + k + 2, b)

            @pl.when(k >= 2)
            def _():
                wait_out(b)

            transpose(b)
            start_out(u0 + k, b)
        return carry

    lax.fori_loop(0, U_PER_W // 2, outer, 0)
    wait_out(0)
    wait_out(1)


def kernel(indices, weight):
    wpad = jnp.concatenate(
        [weight, jnp.zeros((VPAD, DIM), jnp.float32)], axis=0
    )
    wphys = wpad.reshape(VT, 128, 2, 8).transpose(2, 0, 3, 1)
    w_lin = _wt_kernel(wphys)
    idx_phys = (
        indices.reshape(BT, 128, TT, 8)
        .transpose(2, 0, 3, 1)
        .reshape(TT, BT, 1024)
    )
    out_phys = _gather_kernel(w_lin, idx_phys)
    return out_phys.transpose(2, 4, 0, 1, 3).reshape(BATCH, HIST, DIM)
